# baseline (speedup 1.0000x reference)
import jax
import jax.numpy as jnp
from jax import lax
from jax.experimental import pallas as pl
from jax.experimental.pallas import tpu as pltpu
from jax.experimental.pallas import tpu_sc as plsc

N_ROWS = 10000
D = 128
DW = D // 2
E = 320000

NC = 2
NS = 16
NW = NC * NS

EW = E // NW
SB = 2000
NSB = EW // SB
CB = 80
NCHUNK = SB // CB


def _edge_partial(ra, rb, b, e):
    parts = []
    for k in range(DW // 16):
        wa = ra[b, e, pl.ds(k * 16, 16)]
        wb = rb[b, e, pl.ds(k * 16, 16)]
        pa = plsc.bitcast(wa, jnp.bfloat16)
        pb = plsc.bitcast(wb, jnp.bfloat16)
        p0, p1 = plsc.unpack(pa * pb, format=plsc.PackFormat.INTERLEAVED)
        parts.append(p0 + p1)
    return (parts[0] + parts[1]) + (parts[2] + parts[3])


def _dot_chunk(ra, rb, trans, out_v, b, out_base):
    lanes = lax.iota(jnp.int32, 16)
    row_base = lanes * 16

    def group(g, _):
        gbase = g * 16

        def quad(m, _):
            for jj in range(4):
                j = m * 4 + jj
                trans[pl.ds(j * 16, 16)] = _edge_partial(ra, rb, b, gbase + j)
            return 0

        lax.fori_loop(0, 4, quad, 0, unroll=False)

        cols = [plsc.load_gather(trans, [row_base + c]) for c in range(16)]
        while len(cols) > 1:
            cols = [cols[m] + cols[m + 1] for m in range(0, len(cols), 2)]
        out_v[pl.ds(out_base + gbase, 16)] = cols[0]
        return 0

    lax.fori_loop(0, CB // 16, group, 0, unroll=False)


def _kernel_body(xa_hbm, xb_hbm, src_hbm, dst_hbm, out_hbm,
                 sa, ia, ib, ra, rb, trans, out_v, sems):
    cid = lax.axis_index("c")
    sid = lax.axis_index("s")
    wid = sid * NC + cid
    wbase = wid * EW

    @pl.when(sid == 0)
    def _():
        pltpu.sync_copy(xa_hbm, sa)

    plsc.subcore_barrier()

    def gather(c):
        b = lax.rem(c, 2)
        off = c * CB
        pltpu.make_async_copy(
            sa.at[ia.at[pl.ds(off, CB)]], ra.at[b], sems.at[b, 0]).start()
        pltpu.make_async_copy(
            xb_hbm.at[ib.at[pl.ds(off, CB)]], rb.at[b], sems.at[b, 1]).start()

    def wait_chunk(b):
        pltpu.make_async_copy(sa.at[ia.at[pl.ds(0, CB)]],
                              ra.at[b], sems.at[b, 0]).wait()
        pltpu.make_async_copy(xb_hbm.at[ib.at[pl.ds(0, CB)]],
                              rb.at[b], sems.at[b, 1]).wait()

    def super_block(t, _):
        sbase = wbase + t * SB
        pltpu.sync_copy(src_hbm.at[pl.ds(sbase, SB)], ia)
        pltpu.sync_copy(dst_hbm.at[pl.ds(sbase, SB)], ib)

        def step(i, _):
            @pl.when(i < NCHUNK)
            def _():
                gather(i)

            @pl.when(i >= 1)
            def _():
                c = i - 1
                b = lax.rem(c, 2)
                wait_chunk(b)
                _dot_chunk(ra, rb, trans, out_v, b, c * CB)

            return 0

        lax.fori_loop(0, NCHUNK + 1, step, 0, unroll=False)

        pltpu.sync_copy(out_v, out_hbm.at[pl.ds(sbase, SB)])
        return 0

    lax.fori_loop(0, NSB, super_block, 0, unroll=False)


@jax.jit
def _run(xa32, xb32, src, dst):
    mesh = plsc.VectorSubcoreMesh(core_axis_name="c", subcore_axis_name="s")
    return pl.kernel(
        _kernel_body,
        out_type=jax.ShapeDtypeStruct((E,), jnp.float32),
        mesh=mesh,
        compiler_params=pltpu.CompilerParams(needs_layout_passes=False,
                                             use_tc_tiling_on_sc=False),
        scratch_types=[
            pltpu.VMEM_SHARED((N_ROWS, DW), jnp.int32),
            pltpu.VMEM((SB,), jnp.int32),
            pltpu.VMEM((SB,), jnp.int32),
            pltpu.VMEM((2, CB, DW), jnp.int32),
            pltpu.VMEM((2, CB, DW), jnp.int32),
            pltpu.VMEM((256,), jnp.float32),
            pltpu.VMEM((SB,), jnp.float32),
            pltpu.SemaphoreType.DMA((2, 2)),
        ],
    )(xa32, xb32, src, dst)


def _pack_bf16(x):
    return lax.bitcast_convert_type(
        x.astype(jnp.bfloat16).reshape(N_ROWS, DW, 2), jnp.int32)


def kernel(x_miRNA, x_disease, edge_label_index):
    edges = edge_label_index.astype(jnp.int32)
    return _run(_pack_bf16(x_miRNA), _pack_bf16(x_disease),
                edges[0], edges[1])


# speedup vs baseline: 1.5526x; 1.5526x over previous
import jax
import jax.numpy as jnp
from jax import lax
from jax.experimental import pallas as pl
from jax.experimental.pallas import tpu as pltpu
from jax.experimental.pallas import tpu_sc as plsc

N_ROWS = 10000
D = 128
DW = D // 2
E = 320000

NC = 2
NS = 16
NW = NC * NS

EW = E // NW
SB = 2000
NSB = EW // SB
CB = 80
NCHUNK = SB // CB


def _dot_chunk(ra, rb, out_v, b, out_base):
    lanes = lax.iota(jnp.int32, 16)

    def group(g, _):
        gbase = g * 16

        def quad(m, out16):
            for jj in range(4):
                j = m * 4 + jj
                e = gbase + j
                prods = []
                for k in range(DW // 16):
                    wa = ra[b, e, pl.ds(k * 16, 16)]
                    wb = rb[b, e, pl.ds(k * 16, 16)]
                    pa = plsc.bitcast(wa, jnp.bfloat16)
                    pb = plsc.bitcast(wb, jnp.bfloat16)
                    p0, p1 = plsc.unpack(pa * pb,
                                         format=plsc.PackFormat.INTERLEAVED)
                    prods.append(p0 + p1)
                s = jnp.sum((prods[0] + prods[1]) + (prods[2] + prods[3]))
                out16 = jnp.where(lanes == j, s, out16)
            return out16

        out16 = lax.fori_loop(0, 4, quad, jnp.zeros((16,), jnp.float32),
                              unroll=False)
        out_v[pl.ds(out_base + gbase, 16)] = out16
        return 0

    lax.fori_loop(0, CB // 16, group, 0, unroll=False)


def _kernel_body(xa_hbm, xb_hbm, src_hbm, dst_hbm, out_hbm,
                 sa, sb, ia, ib, ra, rb, out_v, sems):
    cid = lax.axis_index("c")
    sid = lax.axis_index("s")
    wid = sid * NC + cid
    wbase = wid * EW

    @pl.when(sid == 0)
    def _():
        pltpu.sync_copy(xa_hbm, sa)
        pltpu.sync_copy(xb_hbm, sb)

    plsc.subcore_barrier()

    def gather(c):
        b = lax.rem(c, 2)
        off = c * CB
        pltpu.make_async_copy(
            sa.at[ia.at[pl.ds(off, CB)]], ra.at[b], sems.at[b]).start()
        pltpu.make_async_copy(
            sb.at[ib.at[pl.ds(off, CB)]], rb.at[b], sems.at[b]).start()

    def wait_chunk(b):
        pltpu.make_async_copy(sa.at[ia.at[pl.ds(0, CB)]],
                              ra.at[b], sems.at[b]).wait()
        pltpu.make_async_copy(sb.at[ib.at[pl.ds(0, CB)]],
                              rb.at[b], sems.at[b]).wait()

    def super_block(t, _):
        sbase = wbase + t * SB
        pltpu.sync_copy(src_hbm.at[pl.ds(sbase, SB)], ia)
        pltpu.sync_copy(dst_hbm.at[pl.ds(sbase, SB)], ib)

        def step(i, _):
            @pl.when(i < NCHUNK)
            def _():
                gather(i)

            @pl.when(i >= 1)
            def _():
                c = i - 1
                b = lax.rem(c, 2)
                wait_chunk(b)
                _dot_chunk(ra, rb, out_v, b, c * CB)

            return 0

        lax.fori_loop(0, NCHUNK + 1, step, 0, unroll=False)

        pltpu.sync_copy(out_v, out_hbm.at[pl.ds(sbase, SB)])
        return 0

    lax.fori_loop(0, NSB, super_block, 0, unroll=False)


@jax.jit
def _run(xa32, xb32, src, dst):
    mesh = plsc.VectorSubcoreMesh(core_axis_name="c", subcore_axis_name="s")
    return pl.kernel(
        _kernel_body,
        out_type=jax.ShapeDtypeStruct((E,), jnp.float32),
        mesh=mesh,
        compiler_params=pltpu.CompilerParams(needs_layout_passes=False,
                                             use_tc_tiling_on_sc=False),
        scratch_types=[
            pltpu.VMEM_SHARED((N_ROWS, DW), jnp.int32),
            pltpu.VMEM_SHARED((N_ROWS, DW), jnp.int32),
            pltpu.VMEM((SB,), jnp.int32),
            pltpu.VMEM((SB,), jnp.int32),
            pltpu.VMEM((2, CB, DW), jnp.int32),
            pltpu.VMEM((2, CB, DW), jnp.int32),
            pltpu.VMEM((SB,), jnp.float32),
            pltpu.SemaphoreType.DMA((2,)),
        ],
    )(xa32, xb32, src, dst)


def _pack_bf16(x):
    return lax.bitcast_convert_type(
        x.astype(jnp.bfloat16).reshape(N_ROWS, DW, 2), jnp.int32)


def kernel(x_miRNA, x_disease, edge_label_index):
    edges = edge_label_index.astype(jnp.int32)
    return _run(_pack_bf16(x_miRNA), _pack_bf16(x_disease),
                edges[0], edges[1])


# speedup vs baseline: 1.6269x; 1.0479x over previous
import jax
import jax.numpy as jnp
from jax import lax
from jax.experimental import pallas as pl
from jax.experimental.pallas import tpu as pltpu
from jax.experimental.pallas import tpu_sc as plsc

N_ROWS = 10000
D = 128
DW = D // 2
E = 320000

NC = 2
NS = 16
NW = NC * NS

EW = E // NW
SB = 10000
NSB = EW // SB
CB = 80
NCHUNK = SB // CB


def _dot_chunk(ra, rb, out_v, b, out_base):
    lanes = lax.iota(jnp.int32, 16)

    def group(g, _):
        gbase = g * 16

        def quad(m, out16):
            for jj in range(4):
                j = m * 4 + jj
                e = gbase + j
                prods = []
                for k in range(DW // 16):
                    wa = ra[b, e, pl.ds(k * 16, 16)]
                    wb = rb[b, e, pl.ds(k * 16, 16)]
                    pa = plsc.bitcast(wa, jnp.bfloat16)
                    pb = plsc.bitcast(wb, jnp.bfloat16)
                    p0, p1 = plsc.unpack(pa * pb,
                                         format=plsc.PackFormat.INTERLEAVED)
                    prods.append(p0 + p1)
                s = jnp.sum((prods[0] + prods[1]) + (prods[2] + prods[3]))
                out16 = jnp.where(lanes == j, s, out16)
            return out16

        out16 = lax.fori_loop(0, 4, quad, jnp.zeros((16,), jnp.float32),
                              unroll=False)
        out_v[pl.ds(out_base + gbase, 16)] = out16
        return 0

    lax.fori_loop(0, CB // 16, group, 0, unroll=False)


def _kernel_body(xa_hbm, xb_hbm, src_hbm, dst_hbm, out_hbm,
                 sa, sb, ia, ib, ra, rb, out_v, sems):
    cid = lax.axis_index("c")
    sid = lax.axis_index("s")
    wid = sid * NC + cid
    wbase = wid * EW

    @pl.when(sid == 0)
    def _():
        pltpu.sync_copy(xa_hbm, sa)
        pltpu.sync_copy(xb_hbm, sb)

    plsc.subcore_barrier()

    def gather(c):
        b = lax.rem(c, 2)
        off = c * CB
        pltpu.make_async_copy(
            sa.at[ia.at[pl.ds(off, CB)]], ra.at[b], sems.at[b]).start()
        pltpu.make_async_copy(
            sb.at[ib.at[pl.ds(off, CB)]], rb.at[b], sems.at[b]).start()

    def wait_chunk(b):
        pltpu.make_async_copy(sa.at[ia.at[pl.ds(0, CB)]],
                              ra.at[b], sems.at[b]).wait()
        pltpu.make_async_copy(sb.at[ib.at[pl.ds(0, CB)]],
                              rb.at[b], sems.at[b]).wait()

    def super_block(t, _):
        sbase = wbase + t * SB
        pltpu.sync_copy(src_hbm.at[pl.ds(sbase, SB)], ia)
        pltpu.sync_copy(dst_hbm.at[pl.ds(sbase, SB)], ib)

        def step(i, _):
            @pl.when(i < NCHUNK)
            def _():
                gather(i)

            @pl.when(i >= 1)
            def _():
                c = i - 1
                b = lax.rem(c, 2)
                wait_chunk(b)
                _dot_chunk(ra, rb, out_v, b, c * CB)

            return 0

        lax.fori_loop(0, NCHUNK + 1, step, 0, unroll=False)

        pltpu.sync_copy(out_v, out_hbm.at[pl.ds(sbase, SB)])
        return 0

    lax.fori_loop(0, NSB, super_block, 0, unroll=False)


@jax.jit
def _run(xa32, xb32, src, dst):
    mesh = plsc.VectorSubcoreMesh(core_axis_name="c", subcore_axis_name="s")
    return pl.kernel(
        _kernel_body,
        out_type=jax.ShapeDtypeStruct((E,), jnp.float32),
        mesh=mesh,
        compiler_params=pltpu.CompilerParams(needs_layout_passes=False,
                                             use_tc_tiling_on_sc=False),
        scratch_types=[
            pltpu.VMEM_SHARED((N_ROWS, DW), jnp.int32),
            pltpu.VMEM_SHARED((N_ROWS, DW), jnp.int32),
            pltpu.VMEM((SB,), jnp.int32),
            pltpu.VMEM((SB,), jnp.int32),
            pltpu.VMEM((2, CB, DW), jnp.int32),
            pltpu.VMEM((2, CB, DW), jnp.int32),
            pltpu.VMEM((SB,), jnp.float32),
            pltpu.SemaphoreType.DMA((2,)),
        ],
    )(xa32, xb32, src, dst)


def _pack_bf16(x):
    return lax.bitcast_convert_type(
        x.astype(jnp.bfloat16).reshape(N_ROWS, DW, 2), jnp.int32)


def kernel(x_miRNA, x_disease, edge_label_index):
    edges = edge_label_index.astype(jnp.int32)
    return _run(_pack_bf16(x_miRNA), _pack_bf16(x_disease),
                edges[0], edges[1])
